# Initial kernel scaffold; baseline (speedup 1.0000x reference)
import jax
import jax.numpy as jnp
from jax.experimental import pallas as pl


def kernel(x, edge_index, norm_elev, norm_length, norm_geom_1, norm_in_offset, norm_out_offset, W1, b1, W2, b2):
    raise NotImplementedError("write your pallas kernel here")


# speedup vs baseline: 9.4667x; 9.4667x over previous
import functools

import jax
import jax.numpy as jnp
from jax import lax
from jax.experimental import pallas as pl
from jax.experimental.pallas import tpu as pltpu
from jax.experimental.pallas import tpu_sc as plsc

N = 100000
E = 1600000
K1 = 5
K2 = 8
KP = 8
TW = 16
NC = 2
NS = 16
NW = NC * NS
EP = E // NW
CH = 400
CHI = 400

_mesh = plsc.VectorSubcoreMesh(core_axis_name="c", subcore_axis_name="s")
_sc_params = pltpu.CompilerParams(needs_layout_passes=False,
                                  use_tc_tiling_on_sc=False)


@functools.partial(
    pl.kernel,
    mesh=_mesh,
    compiler_params=_sc_params,
    out_type=[
        jax.ShapeDtypeStruct((E, KP), jnp.float32),
        jax.ShapeDtypeStruct((E, KP), jnp.float32),
    ],
    scratch_types=[
        pltpu.VMEM((N,), jnp.float32),
        pltpu.VMEM((CHI,), jnp.int32),
        pltpu.VMEM((CHI,), jnp.int32),
        pltpu.VMEM((CHI,), jnp.float32),
        pltpu.VMEM((CHI,), jnp.float32),
        pltpu.VMEM((CHI,), jnp.float32),
        pltpu.VMEM((KP, 16), jnp.float32),
        pltpu.VMEM((KP, 16), jnp.float32),
        pltpu.VMEM((KP, 16), jnp.float32),
        pltpu.VMEM((KP, 16), jnp.float32),
        pltpu.VMEM((CHI, KP), jnp.float32),
        pltpu.VMEM((CHI, KP), jnp.float32),
    ],
)
def _edge_const_kernel(dst_h, src_h, elev_h, io_h, oo_h, ln_h,
                       w10_h, w11_h, w20_h, w21_h,
                       c1_h, c2_h,
                       elev_v, dst_v, src_v, io_v, oo_v, ln_v,
                       w10_v, w11_v, w20_v, w21_v, c1_v, c2_v):
    cid = lax.axis_index("c")
    sid = lax.axis_index("s")
    wid = sid * NC + cid
    pltpu.sync_copy(elev_h, elev_v)
    pltpu.sync_copy(w10_h, w10_v)
    pltpu.sync_copy(w11_h, w11_v)
    pltpu.sync_copy(w20_h, w20_v)
    pltpu.sync_copy(w21_h, w21_v)
    base0 = wid * EP

    def chunk(i, carry):
        base = base0 + i * CHI
        pltpu.sync_copy(dst_h.at[pl.ds(base, CHI)], dst_v)
        pltpu.sync_copy(src_h.at[pl.ds(base, CHI)], src_v)
        pltpu.sync_copy(io_h.at[pl.ds(base, CHI)], io_v)
        pltpu.sync_copy(oo_h.at[pl.ds(base, CHI)], oo_v)
        pltpu.sync_copy(ln_h.at[pl.ds(base, CHI)], ln_v)

        def grp(g, c):
            off = g * 16
            rows = lax.iota(jnp.int32, 16) + off
            d16 = dst_v[pl.ds(off, 16)]
            s16 = src_v[pl.ds(off, 16)]
            ed = plsc.load_gather(elev_v, [d16])
            es = plsc.load_gather(elev_v, [s16])
            dl = (es + io_v[pl.ds(off, 16)]) - (ed + oo_v[pl.ds(off, 16)])
            ln16 = ln_v[pl.ds(off, 16)]
            for k in range(KP):
                kc = jnp.full((16,), k, jnp.int32)
                c1 = dl * w10_v[k] + ln16 * w11_v[k]
                plsc.store_scatter(c1_v, [rows, kc], c1)
                c2 = dl * w20_v[k] + ln16 * w21_v[k]
                plsc.store_scatter(c2_v, [rows, kc], c2)
            return c

        lax.fori_loop(0, CHI // 16, grp, 0)
        pltpu.sync_copy(c1_v, c1_h.at[pl.ds(base, CHI)])
        pltpu.sync_copy(c2_v, c2_h.at[pl.ds(base, CHI)])
        return carry

    lax.fori_loop(0, EP // CHI, chunk, 0)


def _make_edge_pass(K):
    @functools.partial(
        pl.kernel,
        mesh=_mesh,
        compiler_params=_sc_params,
        out_type=jax.ShapeDtypeStruct((NC, N, KP), jnp.float32),
        scratch_types=[
            pltpu.VMEM_SHARED((N, KP), jnp.float32),
            pltpu.VMEM((CH,), jnp.int32),
            pltpu.VMEM((CH,), jnp.int32),
            pltpu.VMEM((CH,), jnp.float32),
            pltpu.VMEM((CH, TW), jnp.float32),
            pltpu.VMEM((CH, TW), jnp.float32),
            pltpu.VMEM((CH, KP), jnp.float32),
            pltpu.VMEM((CH, KP), jnp.float32),
            pltpu.SemaphoreType.DMA,
            pltpu.SemaphoreType.DMA,
        ],
    )
    def _edge_pass(tab_h, dst_h, src_h, gm_h, ce_h, zeros_h, out_h,
                   acc_sh, dst_v, src_v, gm_v, td_v, ts_v, c_v, msg_v,
                   sem_a, sem_b):
        cid = lax.axis_index("c")
        sid = lax.axis_index("s")
        wid = sid * NC + cid

        @pl.when(sid == 0)
        def _():
            pltpu.sync_copy(zeros_h, acc_sh)

        pltpu.sync_copy(zeros_h.at[pl.ds(0, CH)], msg_v)
        plsc.subcore_barrier()
        base0 = wid * EP

        def chunk(i, carry):
            base = base0 + i * CH
            pltpu.sync_copy(dst_h.at[pl.ds(base, CH)], dst_v)
            pltpu.sync_copy(src_h.at[pl.ds(base, CH)], src_v)
            pltpu.sync_copy(gm_h.at[pl.ds(base, CH)], gm_v)
            pltpu.sync_copy(ce_h.at[pl.ds(base, CH)], c_v)
            cp_a = pltpu.async_copy(tab_h.at[dst_v], td_v, sem_a)
            cp_b = pltpu.async_copy(tab_h.at[src_v], ts_v, sem_b)
            cp_a.wait()
            cp_b.wait()

            def grp(g, c):
                off = g * 16
                rows = lax.iota(jnp.int32, 16) + off
                gm = gm_v[pl.ds(off, 16)]
                for k in range(K):
                    kc = jnp.full((16,), k, jnp.int32)
                    az = plsc.load_gather(td_v, [rows, kc])
                    bz = plsc.load_gather(ts_v, [rows, jnp.full((16,), TW // 2 + k, jnp.int32)])
                    cz = plsc.load_gather(c_v, [rows, kc])
                    z = az + bz + cz
                    e = jnp.exp(jnp.abs(z) * -2.0)
                    t = (1.0 - e) / (1.0 + e)
                    msg = t * jnp.sign(z) * gm
                    plsc.store_scatter(msg_v, [rows, kc], msg)
                return c

            lax.fori_loop(0, CH // 16, grp, 0)
            pltpu.sync_copy(msg_v, acc_sh.at[dst_v], add=True)
            return carry

        lax.fori_loop(0, EP // CH, chunk, 0)
        plsc.subcore_barrier()

        @pl.when(sid == 0)
        def _():
            pltpu.sync_copy(acc_sh, out_h.at[cid])

    return _edge_pass


_edge_pass_k1 = _make_edge_pass(K1)
_edge_pass_k2 = _make_edge_pass(K2)


def _proj_body(x_ref, w_ref, b_ref, o_ref):
    o_ref[...] = (
        jnp.dot(x_ref[...], w_ref[...], preferred_element_type=jnp.float32)
        + b_ref[...]
    )


def _proj(x, w, b):
    kin = x.shape[1]
    bn = 2000
    return pl.pallas_call(
        _proj_body,
        grid=(N // bn,),
        in_specs=[
            pl.BlockSpec((bn, kin), lambda i: (i, 0)),
            pl.BlockSpec((kin, TW), lambda i: (0, 0)),
            pl.BlockSpec((1, TW), lambda i: (0, 0)),
        ],
        out_specs=pl.BlockSpec((bn, TW), lambda i: (i, 0)),
        out_shape=jax.ShapeDtypeStruct((N, TW), jnp.float32),
    )(x, w, b.reshape(1, TW))


def _proj2_body(a0_ref, a1_ref, w_ref, b_ref, o_ref):
    o_ref[...] = (
        jnp.dot(a0_ref[...] + a1_ref[...], w_ref[...],
                preferred_element_type=jnp.float32)
        + b_ref[...]
    )


def _proj_sum(a0, a1, w, b):
    bn = 2000
    return pl.pallas_call(
        _proj2_body,
        grid=(N // bn,),
        in_specs=[
            pl.BlockSpec((bn, KP), lambda i: (i, 0)),
            pl.BlockSpec((bn, KP), lambda i: (i, 0)),
            pl.BlockSpec((KP, TW), lambda i: (0, 0)),
            pl.BlockSpec((1, TW), lambda i: (0, 0)),
        ],
        out_specs=pl.BlockSpec((bn, TW), lambda i: (i, 0)),
        out_shape=jax.ShapeDtypeStruct((N, TW), jnp.float32),
    )(a0, a1, w, b.reshape(1, TW))


def _sum_body(a0_ref, a1_ref, o_ref):
    o_ref[...] = a0_ref[...] + a1_ref[...]


def _sum2(a0, a1):
    bn = 2000
    return pl.pallas_call(
        _sum_body,
        grid=(N // bn,),
        in_specs=[
            pl.BlockSpec((bn, KP), lambda i: (i, 0)),
            pl.BlockSpec((bn, KP), lambda i: (i, 0)),
        ],
        out_specs=pl.BlockSpec((bn, KP), lambda i: (i, 0)),
        out_shape=jax.ShapeDtypeStruct((N, KP), jnp.float32),
    )(a0, a1)


def kernel(x, edge_index, norm_elev, norm_length, norm_geom_1,
           norm_in_offset, norm_out_offset, W1, b1, W2, b2):
    f32 = jnp.float32
    src = edge_index[0]
    dst = edge_index[1]

    w1a = jnp.pad(W1[:24], ((0, 0), (0, KP - K1)))
    w1b = jnp.pad(W1[24:48], ((0, 0), (0, KP - K1)))
    wcat1 = jnp.concatenate([w1a, w1b], axis=1)
    bcat1 = jnp.concatenate([jnp.pad(b1, (0, KP - K1)),
                             jnp.zeros((KP,), f32)])
    wcat2 = jnp.zeros((KP, TW), f32)
    wcat2 = wcat2.at[:K1, :KP].set(W2[:K1])
    wcat2 = wcat2.at[:K1, KP:].set(W2[K1:2 * K1])
    bcat2 = jnp.concatenate([b2, jnp.zeros((KP,), f32)])

    def _splat(v):
        return jnp.broadcast_to(jnp.pad(v, (0, KP - v.shape[0]))[:, None],
                                (KP, 16)).astype(f32)

    w10 = _splat(W1[48])
    w11 = _splat(W1[49])
    w20 = _splat(W2[2 * K1])
    w21 = _splat(W2[2 * K1 + 1])

    c1, c2 = _edge_const_kernel(dst, src, norm_elev, norm_in_offset,
                                norm_out_offset, norm_length,
                                w10, w11, w20, w21)

    zeros = jnp.zeros((N, KP), f32)
    h0 = x[:, :8]
    runoff = x[:, 8:]
    preds = []
    for step in (0, 8, 16):
        xs = jnp.concatenate([h0, runoff[:, step:step + 16]], axis=1)
        tab1 = _proj(xs, wcat1, bcat1)
        o1 = _edge_pass_k1(tab1, dst, src, norm_geom_1, c1, zeros)
        tab2 = _proj_sum(o1[0], o1[1], wcat2, bcat2)
        o2 = _edge_pass_k2(tab2, dst, src, norm_geom_1, c2, zeros)
        y = _sum2(o2[0], o2[1])
        preds.append(y)
        h0 = y
    return jnp.concatenate(preds, axis=1)


# speedup vs baseline: 10.4010x; 1.0987x over previous
import functools

import jax
import jax.numpy as jnp
from jax import lax
from jax.experimental import pallas as pl
from jax.experimental.pallas import tpu as pltpu
from jax.experimental.pallas import tpu_sc as plsc

N = 100000
E = 1600000
K1 = 5
K2 = 8
KP = 8
TW = 16
NC = 2
NS = 16
NW = NC * NS
EP = E // NW
CH = 400
CHI = 400

_mesh = plsc.VectorSubcoreMesh(core_axis_name="c", subcore_axis_name="s")
_sc_params = pltpu.CompilerParams(needs_layout_passes=False,
                                  use_tc_tiling_on_sc=False)


@functools.partial(
    pl.kernel,
    mesh=_mesh,
    compiler_params=_sc_params,
    out_type=[
        jax.ShapeDtypeStruct((E, KP), jnp.float32),
        jax.ShapeDtypeStruct((E, KP), jnp.float32),
    ],
    scratch_types=[
        pltpu.VMEM((N,), jnp.float32),
        pltpu.VMEM((CHI,), jnp.int32),
        pltpu.VMEM((CHI,), jnp.int32),
        pltpu.VMEM((CHI,), jnp.float32),
        pltpu.VMEM((CHI,), jnp.float32),
        pltpu.VMEM((CHI,), jnp.float32),
        pltpu.VMEM((KP, 16), jnp.float32),
        pltpu.VMEM((KP, 16), jnp.float32),
        pltpu.VMEM((KP, 16), jnp.float32),
        pltpu.VMEM((KP, 16), jnp.float32),
        pltpu.VMEM((CHI, KP), jnp.float32),
        pltpu.VMEM((CHI, KP), jnp.float32),
    ],
)
def _edge_const_kernel(dst_h, src_h, elev_h, io_h, oo_h, ln_h,
                       w10_h, w11_h, w20_h, w21_h,
                       c1_h, c2_h,
                       elev_v, dst_v, src_v, io_v, oo_v, ln_v,
                       w10_v, w11_v, w20_v, w21_v, c1_v, c2_v):
    cid = lax.axis_index("c")
    sid = lax.axis_index("s")
    wid = sid * NC + cid
    pltpu.sync_copy(elev_h, elev_v)
    pltpu.sync_copy(w10_h, w10_v)
    pltpu.sync_copy(w11_h, w11_v)
    pltpu.sync_copy(w20_h, w20_v)
    pltpu.sync_copy(w21_h, w21_v)
    base0 = wid * EP

    def chunk(i, carry):
        base = base0 + i * CHI
        pltpu.sync_copy(dst_h.at[pl.ds(base, CHI)], dst_v)
        pltpu.sync_copy(src_h.at[pl.ds(base, CHI)], src_v)
        pltpu.sync_copy(io_h.at[pl.ds(base, CHI)], io_v)
        pltpu.sync_copy(oo_h.at[pl.ds(base, CHI)], oo_v)
        pltpu.sync_copy(ln_h.at[pl.ds(base, CHI)], ln_v)

        def grp(g, c):
            off = g * 16
            rows = lax.iota(jnp.int32, 16) + off
            d16 = dst_v[pl.ds(off, 16)]
            s16 = src_v[pl.ds(off, 16)]
            ed = plsc.load_gather(elev_v, [d16])
            es = plsc.load_gather(elev_v, [s16])
            dl = (es + io_v[pl.ds(off, 16)]) - (ed + oo_v[pl.ds(off, 16)])
            ln16 = ln_v[pl.ds(off, 16)]
            for k in range(KP):
                kc = jnp.full((16,), k, jnp.int32)
                c1 = dl * w10_v[k] + ln16 * w11_v[k]
                plsc.store_scatter(c1_v, [rows, kc], c1)
                c2 = dl * w20_v[k] + ln16 * w21_v[k]
                plsc.store_scatter(c2_v, [rows, kc], c2)
            return c

        lax.fori_loop(0, CHI // 16, grp, 0)
        pltpu.sync_copy(c1_v, c1_h.at[pl.ds(base, CHI)])
        pltpu.sync_copy(c2_v, c2_h.at[pl.ds(base, CHI)])
        return carry

    lax.fori_loop(0, EP // CHI, chunk, 0)


def _make_edge_pass(K):
    NCH = EP // CH
    npair = (NCH + 1) // 2
    slot_types = [
        pltpu.VMEM((CH,), jnp.int32),
        pltpu.VMEM((CH,), jnp.int32),
        pltpu.VMEM((CH,), jnp.float32),
        pltpu.VMEM((CH, KP), jnp.float32),
        pltpu.VMEM((CH, TW), jnp.float32),
        pltpu.VMEM((CH, TW), jnp.float32),
        pltpu.VMEM((CH, KP), jnp.float32),
        pltpu.SemaphoreType.DMA,
        pltpu.SemaphoreType.DMA,
    ]

    @functools.partial(
        pl.kernel,
        mesh=_mesh,
        compiler_params=_sc_params,
        out_type=jax.ShapeDtypeStruct((NC, N, KP), jnp.float32),
        scratch_types=[pltpu.VMEM_SHARED((N, KP), jnp.float32)]
        + slot_types + slot_types,
    )
    def _edge_pass(tab_h, dst_h, src_h, gm_h, ce_h, zeros_h, out_h,
                   acc_sh, *slots):
        s0 = slots[:9]
        s1 = slots[9:]
        cid = lax.axis_index("c")
        sid = lax.axis_index("s")
        wid = sid * NC + cid

        @pl.when(sid == 0)
        def _():
            pltpu.sync_copy(zeros_h, acc_sh)

        pltpu.sync_copy(zeros_h.at[pl.ds(0, CH)], s0[6])
        pltpu.sync_copy(zeros_h.at[pl.ds(0, CH)], s1[6])
        plsc.subcore_barrier()
        base0 = wid * EP

        def start(c, S):
            base = base0 + c * CH
            pltpu.sync_copy(dst_h.at[pl.ds(base, CH)], S[0])
            pltpu.sync_copy(src_h.at[pl.ds(base, CH)], S[1])
            pltpu.sync_copy(gm_h.at[pl.ds(base, CH)], S[2])
            pltpu.sync_copy(ce_h.at[pl.ds(base, CH)], S[3])
            pltpu.async_copy(tab_h.at[S[0]], S[4], S[7])
            pltpu.async_copy(tab_h.at[S[1]], S[5], S[8])

        def finish(S):
            pltpu.make_async_copy(tab_h.at[S[0]], S[4], S[7]).wait()
            pltpu.make_async_copy(tab_h.at[S[1]], S[5], S[8]).wait()
            td_v, ts_v, msg_v = S[4], S[5], S[6]

            def grp(g, carry):
                off = g * 16
                rows = lax.iota(jnp.int32, 16) + off
                gm = S[2][pl.ds(off, 16)]
                for k in range(K):
                    kc = jnp.full((16,), k, jnp.int32)
                    az = plsc.load_gather(td_v, [rows, kc])
                    bz = plsc.load_gather(ts_v, [rows, jnp.full((16,), TW // 2 + k, jnp.int32)])
                    cz = plsc.load_gather(S[3], [rows, kc])
                    z = az + bz + cz
                    e = jnp.exp(jnp.abs(z) * -2.0)
                    t = (1.0 - e) / (1.0 + e)
                    msg = t * jnp.sign(z) * gm
                    plsc.store_scatter(msg_v, [rows, kc], msg)
                return carry

            lax.fori_loop(0, CH // 16, grp, 0)
            pltpu.sync_copy(msg_v, acc_sh.at[S[0]], add=True)

        start(0, s0)

        def pair(j, carry):
            c1 = 2 * j + 1
            c2 = 2 * j + 2

            @pl.when(c1 < NCH)
            def _():
                start(c1, s1)

            finish(s0)

            @pl.when(c2 < NCH)
            def _():
                start(c2, s0)

            @pl.when(c1 < NCH)
            def _():
                finish(s1)

            return carry

        lax.fori_loop(0, npair, pair, 0)
        plsc.subcore_barrier()

        @pl.when(sid == 0)
        def _():
            pltpu.sync_copy(acc_sh, out_h.at[cid])

    return _edge_pass


_edge_pass_k1 = _make_edge_pass(K1)
_edge_pass_k2 = _make_edge_pass(K2)


def _proj_body(x_ref, w_ref, b_ref, o_ref):
    o_ref[...] = (
        jnp.dot(x_ref[...], w_ref[...], preferred_element_type=jnp.float32)
        + b_ref[...]
    )


def _proj(x, w, b):
    kin = x.shape[1]
    bn = 2000
    return pl.pallas_call(
        _proj_body,
        grid=(N // bn,),
        in_specs=[
            pl.BlockSpec((bn, kin), lambda i: (i, 0)),
            pl.BlockSpec((kin, TW), lambda i: (0, 0)),
            pl.BlockSpec((1, TW), lambda i: (0, 0)),
        ],
        out_specs=pl.BlockSpec((bn, TW), lambda i: (i, 0)),
        out_shape=jax.ShapeDtypeStruct((N, TW), jnp.float32),
    )(x, w, b.reshape(1, TW))


def _proj2_body(a0_ref, a1_ref, w_ref, b_ref, o_ref):
    o_ref[...] = (
        jnp.dot(a0_ref[...] + a1_ref[...], w_ref[...],
                preferred_element_type=jnp.float32)
        + b_ref[...]
    )


def _proj_sum(a0, a1, w, b):
    bn = 2000
    return pl.pallas_call(
        _proj2_body,
        grid=(N // bn,),
        in_specs=[
            pl.BlockSpec((bn, KP), lambda i: (i, 0)),
            pl.BlockSpec((bn, KP), lambda i: (i, 0)),
            pl.BlockSpec((KP, TW), lambda i: (0, 0)),
            pl.BlockSpec((1, TW), lambda i: (0, 0)),
        ],
        out_specs=pl.BlockSpec((bn, TW), lambda i: (i, 0)),
        out_shape=jax.ShapeDtypeStruct((N, TW), jnp.float32),
    )(a0, a1, w, b.reshape(1, TW))


def _sum_body(a0_ref, a1_ref, o_ref):
    o_ref[...] = a0_ref[...] + a1_ref[...]


def _sum2(a0, a1):
    bn = 2000
    return pl.pallas_call(
        _sum_body,
        grid=(N // bn,),
        in_specs=[
            pl.BlockSpec((bn, KP), lambda i: (i, 0)),
            pl.BlockSpec((bn, KP), lambda i: (i, 0)),
        ],
        out_specs=pl.BlockSpec((bn, KP), lambda i: (i, 0)),
        out_shape=jax.ShapeDtypeStruct((N, KP), jnp.float32),
    )(a0, a1)


def kernel(x, edge_index, norm_elev, norm_length, norm_geom_1,
           norm_in_offset, norm_out_offset, W1, b1, W2, b2):
    f32 = jnp.float32
    src = edge_index[0]
    dst = edge_index[1]

    w1a = jnp.pad(W1[:24], ((0, 0), (0, KP - K1)))
    w1b = jnp.pad(W1[24:48], ((0, 0), (0, KP - K1)))
    wcat1 = jnp.concatenate([w1a, w1b], axis=1)
    bcat1 = jnp.concatenate([jnp.pad(b1, (0, KP - K1)),
                             jnp.zeros((KP,), f32)])
    wcat2 = jnp.zeros((KP, TW), f32)
    wcat2 = wcat2.at[:K1, :KP].set(W2[:K1])
    wcat2 = wcat2.at[:K1, KP:].set(W2[K1:2 * K1])
    bcat2 = jnp.concatenate([b2, jnp.zeros((KP,), f32)])

    def _splat(v):
        return jnp.broadcast_to(jnp.pad(v, (0, KP - v.shape[0]))[:, None],
                                (KP, 16)).astype(f32)

    w10 = _splat(W1[48])
    w11 = _splat(W1[49])
    w20 = _splat(W2[2 * K1])
    w21 = _splat(W2[2 * K1 + 1])

    c1, c2 = _edge_const_kernel(dst, src, norm_elev, norm_in_offset,
                                norm_out_offset, norm_length,
                                w10, w11, w20, w21)

    zeros = jnp.zeros((N, KP), f32)
    h0 = x[:, :8]
    runoff = x[:, 8:]
    preds = []
    for step in (0, 8, 16):
        xs = jnp.concatenate([h0, runoff[:, step:step + 16]], axis=1)
        tab1 = _proj(xs, wcat1, bcat1)
        o1 = _edge_pass_k1(tab1, dst, src, norm_geom_1, c1, zeros)
        tab2 = _proj_sum(o1[0], o1[1], wcat2, bcat2)
        o2 = _edge_pass_k2(tab2, dst, src, norm_geom_1, c2, zeros)
        y = _sum2(o2[0], o2[1])
        preds.append(y)
        h0 = y
    return jnp.concatenate(preds, axis=1)
